# Initial kernel scaffold; baseline (speedup 1.0000x reference)
import jax
import jax.numpy as jnp
from jax.experimental import pallas as pl


def kernel(x, edge_index, batch, W1, b1, W2, b2, W_lin, b_lin):
    raise NotImplementedError("write your pallas kernel here")


# speedup vs baseline: 12.0872x; 12.0872x over previous
import functools

import jax
import jax.numpy as jnp
from jax import lax
from jax.experimental import pallas as pl
from jax.experimental.pallas import tpu as pltpu
from jax.experimental.pallas import tpu_sc as plsc

N_NODES = 10000
N_EDGES = 320000
D_FEAT = 128
HIDDEN = 128
N_CLASSES = 16
N_GRAPHS = 64

NC = 2
NS = 16
NW = NC * NS
EPT = N_EDGES // NW
CH = 80
NCHUNK = EPT // CH
N_PAD = 10240
RPT = N_PAD // NS

def _deg_body(dst_hbm, zeros_hbm, ones_hbm, out_hbm, didx, ones_v, shared, sem):
    c = lax.axis_index("c")
    s = lax.axis_index("s")
    wid = s * NC + c
    pltpu.sync_copy(zeros_hbm, shared.at[pl.ds(s * RPT, RPT)])
    pltpu.sync_copy(ones_hbm, ones_v)
    plsc.subcore_barrier()

    def body(j, carry):
        start = wid * EPT + j * CH
        pltpu.sync_copy(dst_hbm.at[pl.ds(start, CH)], didx)
        pltpu.sync_copy(ones_v, shared.at[didx], add=True)
        return carry

    lax.fori_loop(0, NCHUNK, body, 0)
    plsc.subcore_barrier()
    pltpu.sync_copy(shared.at[pl.ds(s * RPT, RPT)],
                    out_hbm.at[c, pl.ds(s * RPT, RPT)])


def _agg_body(g_hbm, src_hbm, dst_hbm, zeros_hbm, out_hbm,
              sidx, didx, rows, shared, sem):
    c = lax.axis_index("c")
    s = lax.axis_index("s")
    wid = s * NC + c
    pltpu.sync_copy(zeros_hbm, shared.at[pl.ds(s * RPT, RPT)])
    plsc.subcore_barrier()

    def body(j, carry):
        start = wid * EPT + j * CH
        pltpu.sync_copy(src_hbm.at[pl.ds(start, CH)], sidx)
        pltpu.sync_copy(dst_hbm.at[pl.ds(start, CH)], didx)
        pltpu.async_copy(g_hbm.at[sidx], rows, sem).wait()
        pltpu.sync_copy(rows, shared.at[didx], add=True)
        return carry

    lax.fori_loop(0, NCHUNK, body, 0)
    plsc.subcore_barrier()
    pltpu.sync_copy(shared.at[pl.ds(s * RPT, RPT)],
                    out_hbm.at[c, pl.ds(s * RPT, RPT)])


@functools.cache
def _build_sc_kernels():
    mesh = plsc.VectorSubcoreMesh(core_axis_name="c", subcore_axis_name="s")
    deg = pl.kernel(
        _deg_body,
        out_type=jax.ShapeDtypeStruct((NC, N_PAD, HIDDEN), jnp.float32),
        mesh=mesh,
        scratch_types=[
            pltpu.VMEM((CH,), jnp.int32),
            pltpu.VMEM((CH, HIDDEN), jnp.float32),
            pltpu.VMEM_SHARED((N_PAD, HIDDEN), jnp.float32),
            pltpu.SemaphoreType.DMA,
        ],
    )
    agg = pl.kernel(
        _agg_body,
        out_type=jax.ShapeDtypeStruct((NC, N_PAD, HIDDEN), jnp.float32),
        mesh=mesh,
        scratch_types=[
            pltpu.VMEM((CH,), jnp.int32),
            pltpu.VMEM((CH,), jnp.int32),
            pltpu.VMEM((CH, HIDDEN), jnp.float32),
            pltpu.VMEM_SHARED((N_PAD, HIDDEN), jnp.float32),
            pltpu.SemaphoreType.DMA,
        ],
    )
    return deg, agg


def _deg_kernel(*args):
    return _build_sc_kernels()[0](*args)


def _agg_kernel(*args):
    return _build_sc_kernels()[1](*args)


RB = 1000
GRID = N_NODES // RB


def _dinv_of(deg2):
    deg = deg2[0, :, 0] + deg2[1, :, 0] + 1.0
    return lax.rsqrt(deg)


def _prescale_body(deg2_ref, x_ref, w1_ref, g1_ref):
    dinv = _dinv_of(deg2_ref[...])
    h = jnp.dot(x_ref[...], w1_ref[...], preferred_element_type=jnp.float32)
    g1_ref[...] = h * dinv[:, None]


def _mid_body(deg2_ref, s1_ref, g1_ref, b1_ref, w2_ref, g2_ref):
    dinv = _dinv_of(deg2_ref[...])
    agg = s1_ref[0] + s1_ref[1] + g1_ref[...]
    h1 = jnp.maximum(dinv[:, None] * agg + b1_ref[...], 0.0)
    h2 = jnp.dot(h1, w2_ref[...], preferred_element_type=jnp.float32)
    g2_ref[...] = h2 * dinv[:, None]


def _final_body(lohi_ref, deg2_ref, s2_ref, g2_ref, b2_ref, ids_ref, wl_ref,
                bl_ref, out_ref, acc_ref):
    i = pl.program_id(0)

    @pl.when(i == 0)
    def _():
        acc_ref[...] = jnp.full((N_GRAPHS, HIDDEN), -jnp.inf, jnp.float32)

    dinv = _dinv_of(deg2_ref[...])
    agg = s2_ref[0] + s2_ref[1] + g2_ref[...]
    h2 = dinv[:, None] * agg + b2_ref[...]
    ids2 = ids_ref[...]

    acc = acc_ref[...]
    seg_rows = lax.broadcasted_iota(jnp.int32, (N_GRAPHS, 1), 0)

    def seg_body(sid, acc):
        mask = ids2 == sid
        cand = jnp.max(jnp.where(mask, h2, -jnp.inf), axis=0)
        upd = jnp.maximum(acc, cand[None, :])
        return jnp.where(seg_rows == sid, upd, acc)

    acc = lax.fori_loop(lohi_ref[0, 0, 0], lohi_ref[0, 0, 1] + 1, seg_body, acc)
    acc_ref[...] = acc

    @pl.when(i == GRID - 1)
    def _():
        out_ref[...] = (
            jnp.dot(acc, wl_ref[...], preferred_element_type=jnp.float32)
            + bl_ref[...]
        )


def _deg2_spec():
    return pl.BlockSpec((2, RB, HIDDEN), lambda i: (0, i, 0))


def _row_spec():
    return pl.BlockSpec((RB, HIDDEN), lambda i: (i, 0))


def _part_spec():
    return pl.BlockSpec((2, RB, HIDDEN), lambda i: (0, i, 0))


def _full_spec(shape):
    return pl.BlockSpec(shape, lambda i: tuple(0 for _ in shape))


def kernel(x, edge_index, batch, W1, b1, W2, b2, W_lin, b_lin):
    src = edge_index[0].astype(jnp.int32)
    dst = edge_index[1].astype(jnp.int32)
    zeros128 = jnp.zeros((RPT, HIDDEN), jnp.float32)
    ones128 = jnp.ones((CH, HIDDEN), jnp.float32)
    ids_i32 = batch.astype(jnp.int32)
    ids_col = ids_i32.reshape(N_NODES, 1)
    lohi = jnp.stack(
        [ids_i32.reshape(GRID, RB)[:, 0], ids_i32.reshape(GRID, RB)[:, -1]],
        axis=1).reshape(GRID, 1, 2)

    deg2 = _deg_kernel(dst, zeros128, ones128)

    g1 = pl.pallas_call(
        _prescale_body,
        grid=(GRID,),
        in_specs=[_deg2_spec(),
                  pl.BlockSpec((RB, D_FEAT), lambda i: (i, 0)),
                  _full_spec((D_FEAT, HIDDEN))],
        out_specs=_row_spec(),
        out_shape=jax.ShapeDtypeStruct((N_NODES, HIDDEN), jnp.float32),
    )(deg2, x, W1)

    s1 = _agg_kernel(g1, src, dst, zeros128)

    g2 = pl.pallas_call(
        _mid_body,
        grid=(GRID,),
        in_specs=[_deg2_spec(), _part_spec(), _row_spec(),
                  _full_spec((1, HIDDEN)), _full_spec((HIDDEN, HIDDEN))],
        out_specs=_row_spec(),
        out_shape=jax.ShapeDtypeStruct((N_NODES, HIDDEN), jnp.float32),
    )(deg2, s1, g1, b1.reshape(1, HIDDEN), W2)

    s2 = _agg_kernel(g2, src, dst, zeros128)

    out = pl.pallas_call(
        _final_body,
        grid=(GRID,),
        in_specs=[pl.BlockSpec((1, 1, 2), lambda i: (i, 0, 0),
                               memory_space=pltpu.MemorySpace.SMEM),
                  _deg2_spec(), _part_spec(), _row_spec(),
                  _full_spec((1, HIDDEN)),
                  pl.BlockSpec((RB, 1), lambda i: (i, 0)),
                  _full_spec((HIDDEN, N_CLASSES)),
                  _full_spec((1, N_CLASSES))],
        out_specs=_full_spec((N_GRAPHS, N_CLASSES)),
        out_shape=jax.ShapeDtypeStruct((N_GRAPHS, N_CLASSES), jnp.float32),
        scratch_shapes=[pltpu.VMEM((N_GRAPHS, HIDDEN), jnp.float32)],
        compiler_params=pltpu.CompilerParams(
            dimension_semantics=("arbitrary",)),
    )(lohi, deg2, s2, g2, b2.reshape(1, HIDDEN), ids_col, W_lin,
      b_lin.reshape(1, N_CLASSES))

    return out


# speedup vs baseline: 21.3258x; 1.7643x over previous
import functools

import jax
import jax.numpy as jnp
from jax import lax
from jax.experimental import pallas as pl
from jax.experimental.pallas import tpu as pltpu
from jax.experimental.pallas import tpu_sc as plsc

N_NODES = 10000
N_EDGES = 320000
D_FEAT = 128
HIDDEN = 128
N_CLASSES = 16
N_GRAPHS = 64

NC = 2
NS = 16
NW = NC * NS
EPT = N_EDGES // NW
CH = 80
NCHUNK = EPT // CH
N_PAD = 10240
RPT = N_PAD // NS


def _load_chunk(idx_all, j, dbuf):
    for k in range(CH // 16):
        dbuf[pl.ds(k * 16, 16)] = idx_all[pl.ds(j * CH + k * 16, 16)]


def _deg_body(dst_hbm, zeros_hbm, ones_hbm, out_hbm,
              didx_all, ones_v, didx_a, didx_b, shared, sem_a, sem_b):
    c = lax.axis_index("c")
    s = lax.axis_index("s")
    wid = s * NC + c
    base = wid * EPT
    pltpu.sync_copy(zeros_hbm, shared.at[pl.ds(s * RPT, RPT)])
    pltpu.sync_copy(ones_hbm, ones_v)
    pltpu.sync_copy(dst_hbm.at[pl.ds(base, EPT)], didx_all)
    plsc.subcore_barrier()

    def wait_scatter(sem):
        pltpu.make_async_copy(ones_v, shared.at[pl.ds(0, CH)], sem).wait()

    _load_chunk(didx_all, 0, didx_a)
    pltpu.async_copy(ones_v, shared.at[didx_a], sem_a, add=True)
    _load_chunk(didx_all, 1, didx_b)
    pltpu.async_copy(ones_v, shared.at[didx_b], sem_b, add=True)

    def body(i, carry):
        j = 2 + 2 * i
        wait_scatter(sem_a)
        _load_chunk(didx_all, j, didx_a)
        pltpu.async_copy(ones_v, shared.at[didx_a], sem_a, add=True)
        wait_scatter(sem_b)
        _load_chunk(didx_all, j + 1, didx_b)
        pltpu.async_copy(ones_v, shared.at[didx_b], sem_b, add=True)
        return carry

    lax.fori_loop(0, (NCHUNK - 2) // 2, body, 0)
    wait_scatter(sem_a)
    _load_chunk(didx_all, NCHUNK - 1, didx_a)
    pltpu.async_copy(ones_v, shared.at[didx_a], sem_a, add=True)
    wait_scatter(sem_a)
    wait_scatter(sem_b)
    plsc.subcore_barrier()
    pltpu.sync_copy(shared.at[pl.ds(s * RPT, RPT)],
                    out_hbm.at[c, pl.ds(s * RPT, RPT)])


def _agg_body(src_hbm, dst_hbm, g_hbm, zeros_hbm, out_hbm,
              sidx_all, didx_all, rows_a, rows_b, didx_a, didx_b, shared,
              semg_a, semg_b, sems_a, sems_b):
    c = lax.axis_index("c")
    s = lax.axis_index("s")
    wid = s * NC + c
    base = wid * EPT
    pltpu.sync_copy(zeros_hbm, shared.at[pl.ds(s * RPT, RPT)])
    pltpu.sync_copy(src_hbm.at[pl.ds(base, EPT)], sidx_all)
    pltpu.sync_copy(dst_hbm.at[pl.ds(base, EPT)], didx_all)
    plsc.subcore_barrier()

    def start_gather(j, buf, sem):
        pltpu.async_copy(g_hbm.at[sidx_all.at[pl.ds(j * CH, CH)]], buf, sem)

    def wait_gather(buf, sem):
        pltpu.make_async_copy(g_hbm.at[pl.ds(0, CH)], buf, sem).wait()

    def start_scatter(dbuf, buf, sem):
        pltpu.async_copy(buf, shared.at[dbuf], sem, add=True)

    def wait_scatter(buf, sem):
        pltpu.make_async_copy(buf, shared.at[pl.ds(0, CH)], sem).wait()

    start_gather(0, rows_a, semg_a)
    start_gather(1, rows_b, semg_b)
    wait_gather(rows_a, semg_a)
    _load_chunk(didx_all, 0, didx_a)
    start_scatter(didx_a, rows_a, sems_a)
    wait_gather(rows_b, semg_b)
    _load_chunk(didx_all, 1, didx_b)
    start_scatter(didx_b, rows_b, sems_b)

    def body(i, carry):
        j = 2 + 2 * i
        wait_scatter(rows_a, sems_a)
        start_gather(j, rows_a, semg_a)
        wait_scatter(rows_b, sems_b)
        start_gather(j + 1, rows_b, semg_b)
        wait_gather(rows_a, semg_a)
        _load_chunk(didx_all, j, didx_a)
        start_scatter(didx_a, rows_a, sems_a)
        wait_gather(rows_b, semg_b)
        _load_chunk(didx_all, j + 1, didx_b)
        start_scatter(didx_b, rows_b, sems_b)
        return carry

    lax.fori_loop(0, (NCHUNK - 2) // 2, body, 0)
    wait_scatter(rows_a, sems_a)
    start_gather(NCHUNK - 1, rows_a, semg_a)
    wait_gather(rows_a, semg_a)
    _load_chunk(didx_all, NCHUNK - 1, didx_a)
    start_scatter(didx_a, rows_a, sems_a)
    wait_scatter(rows_a, sems_a)
    wait_scatter(rows_b, sems_b)
    plsc.subcore_barrier()
    pltpu.sync_copy(shared.at[pl.ds(s * RPT, RPT)],
                    out_hbm.at[c, pl.ds(s * RPT, RPT)])


@functools.cache
def _build_sc_kernels():
    mesh = plsc.VectorSubcoreMesh(core_axis_name="c", subcore_axis_name="s")
    deg = pl.kernel(
        _deg_body,
        out_type=jax.ShapeDtypeStruct((NC, N_PAD, HIDDEN), jnp.float32),
        mesh=mesh,
        scratch_types=[
            pltpu.VMEM((EPT,), jnp.int32),
            pltpu.VMEM((CH, HIDDEN), jnp.float32),
            pltpu.VMEM((CH,), jnp.int32),
            pltpu.VMEM((CH,), jnp.int32),
            pltpu.VMEM_SHARED((N_PAD, HIDDEN), jnp.float32),
            pltpu.SemaphoreType.DMA,
            pltpu.SemaphoreType.DMA,
        ],
    )
    agg = pl.kernel(
        _agg_body,
        out_type=jax.ShapeDtypeStruct((NC, N_PAD, HIDDEN), jnp.float32),
        mesh=mesh,
        scratch_types=[
            pltpu.VMEM((EPT,), jnp.int32),
            pltpu.VMEM((EPT,), jnp.int32),
            pltpu.VMEM((CH, HIDDEN), jnp.float32),
            pltpu.VMEM((CH, HIDDEN), jnp.float32),
            pltpu.VMEM((CH,), jnp.int32),
            pltpu.VMEM((CH,), jnp.int32),
            pltpu.VMEM_SHARED((N_PAD, HIDDEN), jnp.float32),
            pltpu.SemaphoreType.DMA,
            pltpu.SemaphoreType.DMA,
            pltpu.SemaphoreType.DMA,
            pltpu.SemaphoreType.DMA,
        ],
    )
    return deg, agg


def _deg_kernel(*args):
    return _build_sc_kernels()[0](*args)


def _agg_kernel(*args):
    return _build_sc_kernels()[1](*args)


RB = 1000
GRID = N_NODES // RB


def _dinv_of(deg2):
    deg = deg2[0, :, 0] + deg2[1, :, 0] + 1.0
    return lax.rsqrt(deg)


def _prescale_body(deg2_ref, x_ref, w1_ref, g1_ref):
    dinv = _dinv_of(deg2_ref[...])
    h = jnp.dot(x_ref[...], w1_ref[...], preferred_element_type=jnp.float32)
    g1_ref[...] = h * dinv[:, None]


def _mid_body(deg2_ref, s1_ref, g1_ref, b1_ref, w2_ref, g2_ref):
    dinv = _dinv_of(deg2_ref[...])
    agg = s1_ref[0] + s1_ref[1] + g1_ref[...]
    h1 = jnp.maximum(dinv[:, None] * agg + b1_ref[...], 0.0)
    h2 = jnp.dot(h1, w2_ref[...], preferred_element_type=jnp.float32)
    g2_ref[...] = h2 * dinv[:, None]


def _final_body(lohi_ref, deg2_ref, s2_ref, g2_ref, b2_ref, ids_ref, wl_ref,
                bl_ref, out_ref, acc_ref):
    i = pl.program_id(0)

    @pl.when(i == 0)
    def _():
        acc_ref[...] = jnp.full((N_GRAPHS, HIDDEN), -jnp.inf, jnp.float32)

    dinv = _dinv_of(deg2_ref[...])
    agg = s2_ref[0] + s2_ref[1] + g2_ref[...]
    h2 = dinv[:, None] * agg + b2_ref[...]
    ids2 = ids_ref[...]

    acc = acc_ref[...]
    seg_rows = lax.broadcasted_iota(jnp.int32, (N_GRAPHS, 1), 0)

    def seg_body(sid, acc):
        mask = ids2 == sid
        cand = jnp.max(jnp.where(mask, h2, -jnp.inf), axis=0)
        upd = jnp.maximum(acc, cand[None, :])
        return jnp.where(seg_rows == sid, upd, acc)

    acc = lax.fori_loop(lohi_ref[0, 0, 0], lohi_ref[0, 0, 1] + 1, seg_body, acc)
    acc_ref[...] = acc

    @pl.when(i == GRID - 1)
    def _():
        out_ref[...] = (
            jnp.dot(acc, wl_ref[...], preferred_element_type=jnp.float32)
            + bl_ref[...]
        )


def _deg2_spec():
    return pl.BlockSpec((2, RB, HIDDEN), lambda i: (0, i, 0))


def _row_spec():
    return pl.BlockSpec((RB, HIDDEN), lambda i: (i, 0))


def _part_spec():
    return pl.BlockSpec((2, RB, HIDDEN), lambda i: (0, i, 0))


def _full_spec(shape):
    return pl.BlockSpec(shape, lambda i: tuple(0 for _ in shape))


def kernel(x, edge_index, batch, W1, b1, W2, b2, W_lin, b_lin):
    src = edge_index[0].astype(jnp.int32)
    dst = edge_index[1].astype(jnp.int32)
    zeros128 = jnp.zeros((RPT, HIDDEN), jnp.float32)
    ones128 = jnp.ones((CH, HIDDEN), jnp.float32)
    ids_i32 = batch.astype(jnp.int32)
    ids_col = ids_i32.reshape(N_NODES, 1)
    lohi = jnp.stack(
        [ids_i32.reshape(GRID, RB)[:, 0], ids_i32.reshape(GRID, RB)[:, -1]],
        axis=1).reshape(GRID, 1, 2)

    deg2 = _deg_kernel(dst, zeros128, ones128)

    g1 = pl.pallas_call(
        _prescale_body,
        grid=(GRID,),
        in_specs=[_deg2_spec(),
                  pl.BlockSpec((RB, D_FEAT), lambda i: (i, 0)),
                  _full_spec((D_FEAT, HIDDEN))],
        out_specs=_row_spec(),
        out_shape=jax.ShapeDtypeStruct((N_NODES, HIDDEN), jnp.float32),
    )(deg2, x, W1)

    s1 = _agg_kernel(src, dst, g1, zeros128)

    g2 = pl.pallas_call(
        _mid_body,
        grid=(GRID,),
        in_specs=[_deg2_spec(), _part_spec(), _row_spec(),
                  _full_spec((1, HIDDEN)), _full_spec((HIDDEN, HIDDEN))],
        out_specs=_row_spec(),
        out_shape=jax.ShapeDtypeStruct((N_NODES, HIDDEN), jnp.float32),
    )(deg2, s1, g1, b1.reshape(1, HIDDEN), W2)

    s2 = _agg_kernel(src, dst, g2, zeros128)

    out = pl.pallas_call(
        _final_body,
        grid=(GRID,),
        in_specs=[pl.BlockSpec((1, 1, 2), lambda i: (i, 0, 0),
                               memory_space=pltpu.MemorySpace.SMEM),
                  _deg2_spec(), _part_spec(), _row_spec(),
                  _full_spec((1, HIDDEN)),
                  pl.BlockSpec((RB, 1), lambda i: (i, 0)),
                  _full_spec((HIDDEN, N_CLASSES)),
                  _full_spec((1, N_CLASSES))],
        out_specs=_full_spec((N_GRAPHS, N_CLASSES)),
        out_shape=jax.ShapeDtypeStruct((N_GRAPHS, N_CLASSES), jnp.float32),
        scratch_shapes=[pltpu.VMEM((N_GRAPHS, HIDDEN), jnp.float32)],
        compiler_params=pltpu.CompilerParams(
            dimension_semantics=("arbitrary",)),
    )(lohi, deg2, s2, g2, b2.reshape(1, HIDDEN), ids_col, W_lin,
      b_lin.reshape(1, N_CLASSES))

    return out


# speedup vs baseline: 21.8299x; 1.0236x over previous
import functools

import jax
import jax.numpy as jnp
from jax import lax
from jax.experimental import pallas as pl
from jax.experimental.pallas import tpu as pltpu
from jax.experimental.pallas import tpu_sc as plsc

N_NODES = 10000
N_EDGES = 320000
D_FEAT = 128
HIDDEN = 128
N_CLASSES = 16
N_GRAPHS = 64

NC = 2
NS = 16
NW = NC * NS
EPT = N_EDGES // NW
N_PAD = 10240
RPT = N_PAD // NS


CH = 104
NCH = EPT // CH
REM = EPT - NCH * CH
NB = 3


def _deg_body(dst_hbm, zeros_hbm, ones_hbm, out_hbm,
              ones_v, didx_0, didx_1, didx_r, shared,
              semi_0, semi_1, sems_0, sems_1):
    c = lax.axis_index("c")
    s = lax.axis_index("s")
    wid = s * NC + c
    base = wid * EPT
    pltpu.sync_copy(zeros_hbm, shared.at[pl.ds(s * RPT, RPT)])
    pltpu.sync_copy(ones_hbm, ones_v)
    plsc.subcore_barrier()

    didx = (didx_0, didx_1)
    semi = (semi_0, semi_1)
    sems = (sems_0, sems_1)

    def start_idx(k, j):
        pltpu.async_copy(dst_hbm.at[pl.ds(base + j * CH, CH)], didx[k],
                         semi[k])

    def wait_idx(k):
        pltpu.make_async_copy(dst_hbm.at[pl.ds(0, CH)], didx[k],
                              semi[k]).wait()

    def start_scatter(k):
        pltpu.async_copy(ones_v, shared.at[didx[k]], sems[k], add=True)

    def wait_scatter(k):
        pltpu.make_async_copy(ones_v, shared.at[pl.ds(0, CH)],
                              sems[k]).wait()

    start_idx(0, 0)
    start_idx(1, 1)

    def body(i, carry):
        j = 2 * i
        wait_idx(0)
        start_scatter(0)
        wait_idx(1)
        start_scatter(1)
        wait_scatter(0)
        start_idx(0, j + 2)
        wait_scatter(1)
        start_idx(1, j + 3)
        return carry

    lax.fori_loop(0, (NCH - 2) // 2, body, 0)
    wait_idx(0)
    start_scatter(0)
    wait_idx(1)
    start_scatter(1)
    pltpu.sync_copy(dst_hbm.at[pl.ds(base + NCH * CH, REM)], didx_r)
    pltpu.async_copy(ones_v.at[pl.ds(0, REM)], shared.at[didx_r], sems_0,
                     add=True)
    wait_scatter(0)
    wait_scatter(1)
    pltpu.make_async_copy(ones_v.at[pl.ds(0, REM)],
                          shared.at[pl.ds(0, REM)], sems_0).wait()
    plsc.subcore_barrier()
    pltpu.sync_copy(shared.at[pl.ds(s * RPT, RPT)],
                    out_hbm.at[c, pl.ds(s * RPT, RPT)])


def _agg_body(src_hbm, dst_hbm, g_hbm, zeros_hbm, out_hbm,
              sidx_0, sidx_1, sidx_2, didx_0, didx_1, didx_2,
              rows_0, rows_1, rows_2, sidx_r, didx_r, rows_r, shared,
              semi_0, semi_1, semi_2,
              semg_0, semg_1, semg_2,
              sems_0, sems_1, sems_2):
    c = lax.axis_index("c")
    s = lax.axis_index("s")
    wid = s * NC + c
    base = wid * EPT
    pltpu.sync_copy(zeros_hbm, shared.at[pl.ds(s * RPT, RPT)])
    plsc.subcore_barrier()

    sidx = (sidx_0, sidx_1, sidx_2)
    didx = (didx_0, didx_1, didx_2)
    rows = (rows_0, rows_1, rows_2)
    semi = (semi_0, semi_1, semi_2)
    semg = (semg_0, semg_1, semg_2)
    sems = (sems_0, sems_1, sems_2)

    def start_idx(k, j):
        pltpu.async_copy(src_hbm.at[pl.ds(base + j * CH, CH)], sidx[k],
                         semi[k])
        pltpu.async_copy(dst_hbm.at[pl.ds(base + j * CH, CH)], didx[k],
                         semi[k])

    def wait_idx(k):
        pltpu.make_async_copy(src_hbm.at[pl.ds(0, CH)], sidx[k],
                              semi[k]).wait()
        pltpu.make_async_copy(dst_hbm.at[pl.ds(0, CH)], didx[k],
                              semi[k]).wait()

    def start_gather(k):
        pltpu.async_copy(g_hbm.at[sidx[k]], rows[k], semg[k])

    def wait_gather(k):
        pltpu.make_async_copy(g_hbm.at[pl.ds(0, CH)], rows[k],
                              semg[k]).wait()

    def start_scatter(k):
        pltpu.async_copy(rows[k], shared.at[didx[k]], sems[k], add=True)

    def wait_scatter(k):
        pltpu.make_async_copy(rows[k], shared.at[pl.ds(0, CH)],
                              sems[k]).wait()

    for k in range(NB):
        start_idx(k, k)
    for k in range(NB):
        wait_idx(k)
        start_gather(k)

    def body(g, carry):
        j = NB * g
        for k in range(NB):
            wait_gather(k)
            start_scatter(k)
        for k in range(NB):
            wait_scatter(k)
            start_idx(k, j + NB + k)
        for k in range(NB):
            wait_idx(k)
            start_gather(k)
        return carry

    lax.fori_loop(0, NCH // NB - 1, body, 0)
    for k in range(NB):
        wait_gather(k)
        start_scatter(k)
    pltpu.sync_copy(src_hbm.at[pl.ds(base + NCH * CH, REM)], sidx_r)
    pltpu.sync_copy(dst_hbm.at[pl.ds(base + NCH * CH, REM)], didx_r)
    pltpu.async_copy(g_hbm.at[sidx_r], rows_r, semg_0)
    pltpu.make_async_copy(g_hbm.at[pl.ds(0, REM)], rows_r, semg_0).wait()
    pltpu.async_copy(rows_r, shared.at[didx_r], sems_0, add=True)
    for k in range(NB):
        wait_scatter(k)
    pltpu.make_async_copy(rows_r, shared.at[pl.ds(0, REM)], sems_0).wait()
    plsc.subcore_barrier()
    pltpu.sync_copy(shared.at[pl.ds(s * RPT, RPT)],
                    out_hbm.at[c, pl.ds(s * RPT, RPT)])


@functools.cache
def _build_sc_kernels():
    mesh = plsc.VectorSubcoreMesh(core_axis_name="c", subcore_axis_name="s")
    deg = pl.kernel(
        _deg_body,
        out_type=jax.ShapeDtypeStruct((NC, N_PAD, HIDDEN), jnp.float32),
        mesh=mesh,
        scratch_types=(
            [pltpu.VMEM((CH, HIDDEN), jnp.float32)]
            + [pltpu.VMEM((CH,), jnp.int32)] * 2
            + [pltpu.VMEM((REM,), jnp.int32)]
            + [pltpu.VMEM_SHARED((N_PAD, HIDDEN), jnp.float32)]
            + [pltpu.SemaphoreType.DMA] * 4
        ),
    )
    agg = pl.kernel(
        _agg_body,
        out_type=jax.ShapeDtypeStruct((NC, N_PAD, HIDDEN), jnp.float32),
        mesh=mesh,
        scratch_types=(
            [pltpu.VMEM((CH,), jnp.int32)] * 6
            + [pltpu.VMEM((CH, HIDDEN), jnp.float32)] * 3
            + [pltpu.VMEM((REM,), jnp.int32)] * 2
            + [pltpu.VMEM((REM, HIDDEN), jnp.float32)]
            + [pltpu.VMEM_SHARED((N_PAD, HIDDEN), jnp.float32)]
            + [pltpu.SemaphoreType.DMA] * 9
        ),
    )
    return deg, agg


def _deg_kernel(*args):
    return _build_sc_kernels()[0](*args)


def _agg_kernel(*args):
    return _build_sc_kernels()[1](*args)


RB = 1000
GRID = N_NODES // RB


def _dinv_of(deg2):
    deg = deg2[0, :, 0] + deg2[1, :, 0] + 1.0
    return lax.rsqrt(deg)


def _prescale_body(deg2_ref, x_ref, w1_ref, g1_ref):
    dinv = _dinv_of(deg2_ref[...])
    h = jnp.dot(x_ref[...], w1_ref[...], preferred_element_type=jnp.float32)
    g1_ref[...] = h * dinv[:, None]


def _mid_body(deg2_ref, s1_ref, g1_ref, b1_ref, w2_ref, g2_ref):
    dinv = _dinv_of(deg2_ref[...])
    agg = s1_ref[0] + s1_ref[1] + g1_ref[...]
    h1 = jnp.maximum(dinv[:, None] * agg + b1_ref[...], 0.0)
    h2 = jnp.dot(h1, w2_ref[...], preferred_element_type=jnp.float32)
    g2_ref[...] = h2 * dinv[:, None]


def _final_body(lohi_ref, deg2_ref, s2_ref, g2_ref, b2_ref, ids_ref, wl_ref,
                bl_ref, out_ref, acc_ref):
    i = pl.program_id(0)

    @pl.when(i == 0)
    def _():
        acc_ref[...] = jnp.full((N_GRAPHS, HIDDEN), -jnp.inf, jnp.float32)

    dinv = _dinv_of(deg2_ref[...])
    agg = s2_ref[0] + s2_ref[1] + g2_ref[...]
    h2 = dinv[:, None] * agg + b2_ref[...]
    ids2 = ids_ref[...]

    acc = acc_ref[...]
    seg_rows = lax.broadcasted_iota(jnp.int32, (N_GRAPHS, 1), 0)

    def seg_body(sid, acc):
        mask = ids2 == sid
        cand = jnp.max(jnp.where(mask, h2, -jnp.inf), axis=0)
        upd = jnp.maximum(acc, cand[None, :])
        return jnp.where(seg_rows == sid, upd, acc)

    acc = lax.fori_loop(lohi_ref[0, 0, 0], lohi_ref[0, 0, 1] + 1, seg_body, acc)
    acc_ref[...] = acc

    @pl.when(i == GRID - 1)
    def _():
        out_ref[...] = (
            jnp.dot(acc, wl_ref[...], preferred_element_type=jnp.float32)
            + bl_ref[...]
        )


def _deg2_spec():
    return pl.BlockSpec((2, RB, HIDDEN), lambda i: (0, i, 0))


def _row_spec():
    return pl.BlockSpec((RB, HIDDEN), lambda i: (i, 0))


def _part_spec():
    return pl.BlockSpec((2, RB, HIDDEN), lambda i: (0, i, 0))


def _full_spec(shape):
    return pl.BlockSpec(shape, lambda i: tuple(0 for _ in shape))


def kernel(x, edge_index, batch, W1, b1, W2, b2, W_lin, b_lin):
    src = edge_index[0].astype(jnp.int32)
    dst = edge_index[1].astype(jnp.int32)
    zeros128 = jnp.zeros((RPT, HIDDEN), jnp.float32)
    ones128 = jnp.ones((CH, HIDDEN), jnp.float32)
    ids_i32 = batch.astype(jnp.int32)
    ids_col = ids_i32.reshape(N_NODES, 1)
    lohi = jnp.stack(
        [ids_i32.reshape(GRID, RB)[:, 0], ids_i32.reshape(GRID, RB)[:, -1]],
        axis=1).reshape(GRID, 1, 2)

    deg2 = _deg_kernel(dst, zeros128, ones128)

    g1 = pl.pallas_call(
        _prescale_body,
        grid=(GRID,),
        in_specs=[_deg2_spec(),
                  pl.BlockSpec((RB, D_FEAT), lambda i: (i, 0)),
                  _full_spec((D_FEAT, HIDDEN))],
        out_specs=_row_spec(),
        out_shape=jax.ShapeDtypeStruct((N_NODES, HIDDEN), jnp.float32),
    )(deg2, x, W1)

    s1 = _agg_kernel(src, dst, g1, zeros128)

    g2 = pl.pallas_call(
        _mid_body,
        grid=(GRID,),
        in_specs=[_deg2_spec(), _part_spec(), _row_spec(),
                  _full_spec((1, HIDDEN)), _full_spec((HIDDEN, HIDDEN))],
        out_specs=_row_spec(),
        out_shape=jax.ShapeDtypeStruct((N_NODES, HIDDEN), jnp.float32),
    )(deg2, s1, g1, b1.reshape(1, HIDDEN), W2)

    s2 = _agg_kernel(src, dst, g2, zeros128)

    out = pl.pallas_call(
        _final_body,
        grid=(GRID,),
        in_specs=[pl.BlockSpec((1, 1, 2), lambda i: (i, 0, 0),
                               memory_space=pltpu.MemorySpace.SMEM),
                  _deg2_spec(), _part_spec(), _row_spec(),
                  _full_spec((1, HIDDEN)),
                  pl.BlockSpec((RB, 1), lambda i: (i, 0)),
                  _full_spec((HIDDEN, N_CLASSES)),
                  _full_spec((1, N_CLASSES))],
        out_specs=_full_spec((N_GRAPHS, N_CLASSES)),
        out_shape=jax.ShapeDtypeStruct((N_GRAPHS, N_CLASSES), jnp.float32),
        scratch_shapes=[pltpu.VMEM((N_GRAPHS, HIDDEN), jnp.float32)],
        compiler_params=pltpu.CompilerParams(
            dimension_semantics=("arbitrary",)),
    )(lohi, deg2, s2, g2, b2.reshape(1, HIDDEN), ids_col, W_lin,
      b_lin.reshape(1, N_CLASSES))

    return out


# speedup vs baseline: 25.9680x; 1.1896x over previous
import functools

import jax
import jax.numpy as jnp
from jax import lax
from jax.experimental import pallas as pl
from jax.experimental.pallas import tpu as pltpu
from jax.experimental.pallas import tpu_sc as plsc

N_NODES = 10000
N_EDGES = 320000
D_FEAT = 128
HIDDEN = 128
N_CLASSES = 16
N_GRAPHS = 64

NC = 2
NS = 16
NW = NC * NS
EPT = N_EDGES // NW
N_PAD = 10240
RPT = N_PAD // NS


CH = 128
NCH = EPT // CH
REM = EPT - NCH * CH
GRP = 6
NGRP = NCH // GRP
GW = GRP * CH


def _copy_didx(gbuf, p, dbuf):
    for k in range(CH // 16):
        dbuf[pl.ds(k * 16, 16)] = gbuf[pl.ds(p * CH + k * 16, 16)]


def _deg_body(dst_hbm, zeros_hbm, ones_hbm, out_hbm,
              ones_v, gd_a, gd_b, didx_0, didx_1, didx_r, shared,
              semi_a, semi_b, sems_0, sems_1):
    c = lax.axis_index("c")
    s = lax.axis_index("s")
    wid = s * NC + c
    base = wid * EPT
    pltpu.sync_copy(zeros_hbm, shared.at[pl.ds(s * RPT, RPT)])
    pltpu.sync_copy(ones_hbm, ones_v)
    plsc.subcore_barrier()

    didx = (didx_0, didx_1)
    sems = (sems_0, sems_1)

    def fetch_group(gbuf, g, sem):
        g = jnp.minimum(g, NGRP - 1)
        pltpu.async_copy(dst_hbm.at[pl.ds(base + g * GW, GW)], gbuf, sem)

    def wait_group(gbuf, sem):
        pltpu.make_async_copy(dst_hbm.at[pl.ds(0, GW)], gbuf, sem).wait()

    def wait_scatter(k):
        pltpu.make_async_copy(ones_v, shared.at[pl.ds(0, CH)],
                              sems[k]).wait()

    def run_group(gbuf, first):
        for p in range(GRP):
            k = p % 2
            if not (first and p < 2):
                wait_scatter(k)
            _copy_didx(gbuf, p, didx[k])
            pltpu.async_copy(ones_v, shared.at[didx[k]], sems[k], add=True)

    fetch_group(gd_a, 0, semi_a)
    fetch_group(gd_b, 1, semi_b)
    wait_group(gd_a, semi_a)
    run_group(gd_a, True)
    fetch_group(gd_a, 2, semi_a)

    def body(i, carry):
        g = 1 + 2 * i
        wait_group(gd_b, semi_b)
        run_group(gd_b, False)
        fetch_group(gd_b, g + 2, semi_b)
        wait_group(gd_a, semi_a)
        run_group(gd_a, False)
        fetch_group(gd_a, g + 3, semi_a)
        return carry

    lax.fori_loop(0, (NGRP - 1) // 2, body, 0)
    wait_group(gd_a, semi_a)
    wait_group(gd_b, semi_b)
    pltpu.sync_copy(dst_hbm.at[pl.ds(base + NCH * CH, REM)], didx_r)
    wait_scatter(0)
    pltpu.async_copy(ones_v.at[pl.ds(0, REM)], shared.at[didx_r], sems_0,
                     add=True)
    wait_scatter(1)
    pltpu.make_async_copy(ones_v.at[pl.ds(0, REM)],
                          shared.at[pl.ds(0, REM)], sems_0).wait()
    plsc.subcore_barrier()
    pltpu.sync_copy(shared.at[pl.ds(s * RPT, RPT)],
                    out_hbm.at[c, pl.ds(s * RPT, RPT)])


def _agg_body(src_hbm, dst_hbm, g_hbm, zeros_hbm, out_hbm,
              gs_a, gd_a, gs_b, gd_b, didx_0, didx_1, rows_0, rows_1,
              sidx_r, didx_r, rows_r, shared,
              semi_a, semi_b, semg_0, semg_1, sems_0, sems_1):
    c = lax.axis_index("c")
    s = lax.axis_index("s")
    wid = s * NC + c
    base = wid * EPT
    pltpu.sync_copy(zeros_hbm, shared.at[pl.ds(s * RPT, RPT)])
    plsc.subcore_barrier()

    didx = (didx_0, didx_1)
    rows = (rows_0, rows_1)
    semg = (semg_0, semg_1)
    sems = (sems_0, sems_1)

    def fetch_group(sbuf, dbuf, g, sem):
        g = jnp.minimum(g, NGRP - 1)
        pltpu.async_copy(src_hbm.at[pl.ds(base + g * GW, GW)], sbuf, sem)
        pltpu.async_copy(dst_hbm.at[pl.ds(base + g * GW, GW)], dbuf, sem)

    def wait_group(sbuf, dbuf, sem):
        pltpu.make_async_copy(src_hbm.at[pl.ds(0, GW)], sbuf, sem).wait()
        pltpu.make_async_copy(dst_hbm.at[pl.ds(0, GW)], dbuf, sem).wait()

    def wait_gather(k):
        pltpu.make_async_copy(g_hbm.at[pl.ds(0, CH)], rows[k],
                              semg[k]).wait()

    def wait_scatter(k):
        pltpu.make_async_copy(rows[k], shared.at[pl.ds(0, CH)],
                              sems[k]).wait()

    def run_group(sbuf, dbuf, first):
        for p in range(GRP):
            k = p % 2
            if not (first and p < 2):
                wait_scatter(k)
            pltpu.async_copy(g_hbm.at[sbuf.at[pl.ds(p * CH, CH)]],
                             rows[k], semg[k])
            if p >= 1:
                kp = (p - 1) % 2
                wait_gather(kp)
                _copy_didx(dbuf, p - 1, didx[kp])
                pltpu.async_copy(rows[kp], shared.at[didx[kp]], sems[kp],
                                 add=True)
        kl = (GRP - 1) % 2
        wait_gather(kl)
        _copy_didx(dbuf, GRP - 1, didx[kl])
        pltpu.async_copy(rows[kl], shared.at[didx[kl]], sems[kl], add=True)

    fetch_group(gs_a, gd_a, 0, semi_a)
    fetch_group(gs_b, gd_b, 1, semi_b)
    wait_group(gs_a, gd_a, semi_a)
    run_group(gs_a, gd_a, True)
    fetch_group(gs_a, gd_a, 2, semi_a)

    def body(i, carry):
        g = 1 + 2 * i
        wait_group(gs_b, gd_b, semi_b)
        run_group(gs_b, gd_b, False)
        fetch_group(gs_b, gd_b, g + 2, semi_b)
        wait_group(gs_a, gd_a, semi_a)
        run_group(gs_a, gd_a, False)
        fetch_group(gs_a, gd_a, g + 3, semi_a)
        return carry

    lax.fori_loop(0, (NGRP - 1) // 2, body, 0)
    wait_group(gs_a, gd_a, semi_a)
    wait_group(gs_b, gd_b, semi_b)
    pltpu.sync_copy(src_hbm.at[pl.ds(base + NCH * CH, REM)], sidx_r)
    pltpu.sync_copy(dst_hbm.at[pl.ds(base + NCH * CH, REM)], didx_r)
    pltpu.async_copy(g_hbm.at[sidx_r], rows_r, semg_0)
    wait_scatter(0)
    wait_scatter(1)
    pltpu.make_async_copy(g_hbm.at[pl.ds(0, REM)], rows_r, semg_0).wait()
    pltpu.async_copy(rows_r, shared.at[didx_r], sems_0, add=True)
    pltpu.make_async_copy(rows_r, shared.at[pl.ds(0, REM)], sems_0).wait()
    plsc.subcore_barrier()
    pltpu.sync_copy(shared.at[pl.ds(s * RPT, RPT)],
                    out_hbm.at[c, pl.ds(s * RPT, RPT)])


@functools.cache
def _build_sc_kernels():
    mesh = plsc.VectorSubcoreMesh(core_axis_name="c", subcore_axis_name="s")
    deg = pl.kernel(
        _deg_body,
        out_type=jax.ShapeDtypeStruct((NC, N_PAD, HIDDEN), jnp.float32),
        mesh=mesh,
        scratch_types=(
            [pltpu.VMEM((CH, HIDDEN), jnp.float32)]
            + [pltpu.VMEM((GW,), jnp.int32)] * 2
            + [pltpu.VMEM((CH,), jnp.int32)] * 2
            + [pltpu.VMEM((REM,), jnp.int32)]
            + [pltpu.VMEM_SHARED((N_PAD, HIDDEN), jnp.float32)]
            + [pltpu.SemaphoreType.DMA] * 4
        ),
    )
    agg = pl.kernel(
        _agg_body,
        out_type=jax.ShapeDtypeStruct((NC, N_PAD, HIDDEN), jnp.float32),
        mesh=mesh,
        scratch_types=(
            [pltpu.VMEM((GW,), jnp.int32)] * 4
            + [pltpu.VMEM((CH,), jnp.int32)] * 2
            + [pltpu.VMEM((CH, HIDDEN), jnp.float32)] * 2
            + [pltpu.VMEM((REM,), jnp.int32)] * 2
            + [pltpu.VMEM((REM, HIDDEN), jnp.float32)]
            + [pltpu.VMEM_SHARED((N_PAD, HIDDEN), jnp.float32)]
            + [pltpu.SemaphoreType.DMA] * 6
        ),
    )
    return deg, agg


def _deg_kernel(*args):
    return _build_sc_kernels()[0](*args)


def _agg_kernel(*args):
    return _build_sc_kernels()[1](*args)


RB = 1000
GRID = N_NODES // RB


def _dinv_of(deg2):
    deg = deg2[0, :, 0] + deg2[1, :, 0] + 1.0
    return lax.rsqrt(deg)


def _prescale_body(deg2_ref, x_ref, w1_ref, g1_ref):
    dinv = _dinv_of(deg2_ref[...])
    h = jnp.dot(x_ref[...], w1_ref[...], preferred_element_type=jnp.float32)
    g1_ref[...] = h * dinv[:, None]


def _mid_body(deg2_ref, s1_ref, g1_ref, b1_ref, w2_ref, g2_ref):
    dinv = _dinv_of(deg2_ref[...])
    agg = s1_ref[0] + s1_ref[1] + g1_ref[...]
    h1 = jnp.maximum(dinv[:, None] * agg + b1_ref[...], 0.0)
    h2 = jnp.dot(h1, w2_ref[...], preferred_element_type=jnp.float32)
    g2_ref[...] = h2 * dinv[:, None]


def _final_body(lohi_ref, deg2_ref, s2_ref, g2_ref, b2_ref, ids_ref, wl_ref,
                bl_ref, out_ref, acc_ref):
    i = pl.program_id(0)

    @pl.when(i == 0)
    def _():
        acc_ref[...] = jnp.full((N_GRAPHS, HIDDEN), -jnp.inf, jnp.float32)

    dinv = _dinv_of(deg2_ref[...])
    agg = s2_ref[0] + s2_ref[1] + g2_ref[...]
    h2 = dinv[:, None] * agg + b2_ref[...]
    ids2 = ids_ref[...]

    acc = acc_ref[...]
    seg_rows = lax.broadcasted_iota(jnp.int32, (N_GRAPHS, 1), 0)

    def seg_body(sid, acc):
        mask = ids2 == sid
        cand = jnp.max(jnp.where(mask, h2, -jnp.inf), axis=0)
        upd = jnp.maximum(acc, cand[None, :])
        return jnp.where(seg_rows == sid, upd, acc)

    acc = lax.fori_loop(lohi_ref[0, 0, 0], lohi_ref[0, 0, 1] + 1, seg_body, acc)
    acc_ref[...] = acc

    @pl.when(i == GRID - 1)
    def _():
        out_ref[...] = (
            jnp.dot(acc, wl_ref[...], preferred_element_type=jnp.float32)
            + bl_ref[...]
        )


def _deg2_spec():
    return pl.BlockSpec((2, RB, HIDDEN), lambda i: (0, i, 0))


def _row_spec():
    return pl.BlockSpec((RB, HIDDEN), lambda i: (i, 0))


def _part_spec():
    return pl.BlockSpec((2, RB, HIDDEN), lambda i: (0, i, 0))


def _full_spec(shape):
    return pl.BlockSpec(shape, lambda i: tuple(0 for _ in shape))


def kernel(x, edge_index, batch, W1, b1, W2, b2, W_lin, b_lin):
    src = edge_index[0].astype(jnp.int32)
    dst = edge_index[1].astype(jnp.int32)
    zeros128 = jnp.zeros((RPT, HIDDEN), jnp.float32)
    ones128 = jnp.ones((CH, HIDDEN), jnp.float32)
    ids_i32 = batch.astype(jnp.int32)
    ids_col = ids_i32.reshape(N_NODES, 1)
    lohi = jnp.stack(
        [ids_i32.reshape(GRID, RB)[:, 0], ids_i32.reshape(GRID, RB)[:, -1]],
        axis=1).reshape(GRID, 1, 2)

    deg2 = _deg_kernel(dst, zeros128, ones128)

    g1 = pl.pallas_call(
        _prescale_body,
        grid=(GRID,),
        in_specs=[_deg2_spec(),
                  pl.BlockSpec((RB, D_FEAT), lambda i: (i, 0)),
                  _full_spec((D_FEAT, HIDDEN))],
        out_specs=_row_spec(),
        out_shape=jax.ShapeDtypeStruct((N_NODES, HIDDEN), jnp.float32),
    )(deg2, x, W1)

    s1 = _agg_kernel(src, dst, g1, zeros128)

    g2 = pl.pallas_call(
        _mid_body,
        grid=(GRID,),
        in_specs=[_deg2_spec(), _part_spec(), _row_spec(),
                  _full_spec((1, HIDDEN)), _full_spec((HIDDEN, HIDDEN))],
        out_specs=_row_spec(),
        out_shape=jax.ShapeDtypeStruct((N_NODES, HIDDEN), jnp.float32),
    )(deg2, s1, g1, b1.reshape(1, HIDDEN), W2)

    s2 = _agg_kernel(src, dst, g2, zeros128)

    out = pl.pallas_call(
        _final_body,
        grid=(GRID,),
        in_specs=[pl.BlockSpec((1, 1, 2), lambda i: (i, 0, 0),
                               memory_space=pltpu.MemorySpace.SMEM),
                  _deg2_spec(), _part_spec(), _row_spec(),
                  _full_spec((1, HIDDEN)),
                  pl.BlockSpec((RB, 1), lambda i: (i, 0)),
                  _full_spec((HIDDEN, N_CLASSES)),
                  _full_spec((1, N_CLASSES))],
        out_specs=_full_spec((N_GRAPHS, N_CLASSES)),
        out_shape=jax.ShapeDtypeStruct((N_GRAPHS, N_CLASSES), jnp.float32),
        scratch_shapes=[pltpu.VMEM((N_GRAPHS, HIDDEN), jnp.float32)],
        compiler_params=pltpu.CompilerParams(
            dimension_semantics=("arbitrary",)),
    )(lohi, deg2, s2, g2, b2.reshape(1, HIDDEN), ids_col, W_lin,
      b_lin.reshape(1, N_CLASSES))

    return out
